# baseline (speedup 1.0000x reference)
import functools

import jax
import jax.numpy as jnp
from jax import lax
from jax.experimental import pallas as pl
from jax.experimental.pallas import tpu as pltpu
from jax.experimental.pallas import tpu_sc as plsc

N_TOKENS = 3276800
VOCAB = 1000
DIM = 32
LANES = 16

_info = plsc.get_sparse_core_info()
_NC, _NS = _info.num_cores, _info.num_subcores
_NW = _NC * _NS

_B_PER_W = N_TOKENS // _NW
_CHUNK = 1024
_STEPS = _B_PER_W // _CHUNK
_RG = N_TOKENS // 128
_RG_W = _B_PER_W // 128
_CG = DIM // 8
_TPC = _CHUNK // 128


def _make_gather():
    mesh = plsc.VectorSubcoreMesh(core_axis_name="c", subcore_axis_name="s")

    @functools.partial(
        pl.kernel,
        mesh=mesh,
        compiler_params=pltpu.CompilerParams(
            needs_layout_passes=False, use_tc_tiling_on_sc=False
        ),
        out_type=jax.ShapeDtypeStruct((_CG, _RG, 8, 128), jnp.float32),
        scratch_types=[
            pltpu.VMEM((VOCAB * DIM,), jnp.float32),
            pltpu.VMEM((_CHUNK,), jnp.int32),
            pltpu.VMEM((_CHUNK,), jnp.int32),
            pltpu.VMEM((_CG, _TPC, 8, 128), jnp.float32),
            pltpu.VMEM((_CG, _TPC, 8, 128), jnp.float32),
            pltpu.SemaphoreType.DMA,
            pltpu.SemaphoreType.DMA,
            pltpu.SemaphoreType.DMA,
            pltpu.SemaphoreType.DMA,
        ],
    )
    def k(tab_hbm, idx_hbm, out_hbm, tab_v, idx_v0, idx_v1, rows_v0, rows_v1,
          isem0, isem1, osem0, osem1):
        wid = lax.axis_index("s") * _NC + lax.axis_index("c")
        base = wid * _B_PER_W

        pltpu.sync_copy(tab_hbm, tab_v)

        idx_b = (idx_v0, idx_v1)
        rows_b = (rows_v0, rows_v1)
        isem = (isem0, isem1)
        osem = (osem0, osem1)

        def idx_start(s, b):
            pltpu.async_copy(
                idx_hbm.at[pl.ds(base + s * _CHUNK, _CHUNK)], idx_b[b], isem[b]
            )

        def idx_wait(b):
            pltpu.make_async_copy(
                idx_hbm.at[pl.ds(base, _CHUNK)], idx_b[b], isem[b]
            ).wait()

        def out_start(s, b):
            b0 = wid * _RG_W + s * _TPC
            pltpu.async_copy(rows_b[b], out_hbm.at[:, pl.ds(b0, _TPC)], osem[b])

        def out_wait(b):
            pltpu.make_async_copy(
                rows_b[b], out_hbm.at[:, pl.ds(wid * _RG_W, _TPC)], osem[b]
            ).wait()

        def compute(idx_ref, rows_ref):
            @plsc.parallel_loop(0, _TPC, unroll=2)
            def col(j):
                for k8 in range(8):
                    t = idx_ref[pl.ds(j * 128 + k8 * LANES, LANES)]
                    for c in range(DIM):
                        vals = plsc.load_gather(tab_v, [t + c * VOCAB])
                        rows_ref[c // 8, j, c % 8, pl.ds(k8 * LANES, LANES)] = vals

        idx_start(0, 0)
        idx_start(1, 1)
        for b in range(2):
            idx_wait(b)
            compute(idx_b[b], rows_b[b])
            out_start(b, b)
            idx_start(b + 2, b)

        def body(si, carry):
            for b in range(2):
                s = 2 * si + b
                idx_wait(b)
                out_wait(b)
                compute(idx_b[b], rows_b[b])
                out_start(s, b)
                idx_start(jnp.minimum(s + 2, _STEPS - 1), b)
            return carry

        lax.fori_loop(1, _STEPS // 2, body, 0)
        for b in range(2):
            out_wait(b)
            idx_wait(b)

    return k


_gather = _make_gather()


def kernel(tokens, embeddings):
    tab_t = embeddings.T.reshape(-1)
    arr4 = _gather(tab_t, tokens)
    return jnp.transpose(arr4, (1, 3, 0, 2)).reshape(N_TOKENS, DIM)


# speedup vs baseline: 2.3416x; 2.3416x over previous
import functools

import jax
import jax.numpy as jnp
from jax import lax
from jax.experimental import pallas as pl
from jax.experimental.pallas import tpu as pltpu
from jax.experimental.pallas import tpu_sc as plsc

N_TOKENS = 3276800
VOCAB = 1000
DIM = 32
LANES = 16

_info = plsc.get_sparse_core_info()
_NC, _NS = _info.num_cores, _info.num_subcores
_NW = _NC * _NS

_B_PER_W = N_TOKENS // _NW
_CHUNK = 1024
_STEPS = _B_PER_W // _CHUNK
_RG = N_TOKENS // 128
_RG_W = _B_PER_W // 128
_CG = DIM // 8
_TPC = _CHUNK // 128


def _make_gather():
    mesh = plsc.VectorSubcoreMesh(core_axis_name="c", subcore_axis_name="s")

    @functools.partial(
        pl.kernel,
        mesh=mesh,
        compiler_params=pltpu.CompilerParams(
            needs_layout_passes=False, use_tc_tiling_on_sc=False
        ),
        out_type=jax.ShapeDtypeStruct((_CG, _RG, 8, 128), jnp.float32),
        scratch_types=[
            pltpu.VMEM((VOCAB * DIM,), jnp.float32),
            pltpu.VMEM((_CHUNK,), jnp.int32),
            pltpu.VMEM((_CHUNK,), jnp.int32),
            pltpu.VMEM((_CG, _TPC, 8, 128), jnp.float32),
            pltpu.VMEM((_CG, _TPC, 8, 128), jnp.float32),
            pltpu.SemaphoreType.DMA,
            pltpu.SemaphoreType.DMA,
            pltpu.SemaphoreType.DMA,
            pltpu.SemaphoreType.DMA,
        ],
    )
    def k(tab_hbm, idx_hbm, out_hbm, tab_v, idx_v0, idx_v1, rows_v0, rows_v1,
          isem0, isem1, osem0, osem1):
        wid = lax.axis_index("s") * _NC + lax.axis_index("c")
        base = wid * _B_PER_W

        pltpu.sync_copy(tab_hbm, tab_v)

        idx_b = (idx_v0, idx_v1)
        rows_b = (rows_v0, rows_v1)
        isem = (isem0, isem1)
        osem = (osem0, osem1)

        def idx_start(s, b):
            pltpu.async_copy(
                idx_hbm.at[pl.ds(base + s * _CHUNK, _CHUNK)], idx_b[b], isem[b]
            )

        def idx_wait(b):
            pltpu.make_async_copy(
                idx_hbm.at[pl.ds(base, _CHUNK)], idx_b[b], isem[b]
            ).wait()

        def out_start(s, b):
            b0 = wid * _RG_W + s * _TPC
            pltpu.async_copy(rows_b[b], out_hbm.at[:, pl.ds(b0, _TPC)], osem[b])

        def out_wait(b):
            pltpu.make_async_copy(
                rows_b[b], out_hbm.at[:, pl.ds(wid * _RG_W, _TPC)], osem[b]
            ).wait()

        def compute(idx_ref, rows_ref):
            @plsc.parallel_loop(0, _CHUNK // LANES, unroll=4)
            def group(g):
                t = idx_ref[pl.ds(g * LANES, LANES)]
                j = g // 8
                lane0 = (g % 8) * LANES
                for c in range(DIM):
                    vals = plsc.load_gather(tab_v, [t + c * VOCAB])
                    rows_ref[c // 8, j, c % 8, pl.ds(lane0, LANES)] = vals

        idx_start(0, 0)
        idx_start(1, 1)
        for b in range(2):
            idx_wait(b)
            compute(idx_b[b], rows_b[b])
            out_start(b, b)
            idx_start(b + 2, b)

        def body(si, carry):
            for b in range(2):
                s = 2 * si + b
                idx_wait(b)
                out_wait(b)
                compute(idx_b[b], rows_b[b])
                out_start(s, b)
                idx_start(jnp.minimum(s + 2, _STEPS - 1), b)
            return carry

        lax.fori_loop(1, _STEPS // 2, body, 0)
        for b in range(2):
            out_wait(b)
            idx_wait(b)

    return k


_gather = _make_gather()


def kernel(tokens, embeddings):
    tab_t = embeddings.T.reshape(-1)
    arr4 = _gather(tab_t, tokens)
    return jnp.transpose(arr4, (1, 3, 0, 2)).reshape(N_TOKENS, DIM)


# speedup vs baseline: 3.8442x; 1.6417x over previous
import functools

import jax
import jax.numpy as jnp
from jax import lax
from jax.experimental import pallas as pl
from jax.experimental.pallas import tpu as pltpu
from jax.experimental.pallas import tpu_sc as plsc

N_TOKENS = 3276800
VOCAB = 1000
DIM = 32
LANES = 16

_info = plsc.get_sparse_core_info()
_NC, _NS = _info.num_cores, _info.num_subcores
_NW = _NC * _NS

_B_PER_W = N_TOKENS // _NW
_CHUNK = 1024
_STEPS = _B_PER_W // _CHUNK
_RG = N_TOKENS // 128
_RG_W = _B_PER_W // 128
_CG = DIM // 8
_TPC = _CHUNK // 128


def _make_gather():
    mesh = plsc.VectorSubcoreMesh(core_axis_name="c", subcore_axis_name="s")

    @functools.partial(
        pl.kernel,
        mesh=mesh,
        compiler_params=pltpu.CompilerParams(
            needs_layout_passes=False, use_tc_tiling_on_sc=False
        ),
        out_type=jax.ShapeDtypeStruct((_CG, _RG, 8, 128), jnp.float32),
        scratch_types=[
            pltpu.VMEM((VOCAB * DIM,), jnp.float32),
            pltpu.VMEM((_CHUNK,), jnp.int32),
            pltpu.VMEM((_CHUNK,), jnp.int32),
            pltpu.VMEM((_CG, _TPC, 8, 128), jnp.float32),
            pltpu.VMEM((_CG, _TPC, 8, 128), jnp.float32),
            pltpu.SemaphoreType.DMA,
            pltpu.SemaphoreType.DMA,
            pltpu.SemaphoreType.DMA,
            pltpu.SemaphoreType.DMA,
        ],
    )
    def k(tab_hbm, idx_hbm, out_hbm, tab_v, idx_v0, idx_v1, rows_v0, rows_v1,
          isem0, isem1, osem0, osem1):
        wid = lax.axis_index("s") * _NC + lax.axis_index("c")
        base = wid * _B_PER_W

        pltpu.sync_copy(tab_hbm, tab_v)

        idx_b = (idx_v0, idx_v1)
        rows_b = (rows_v0, rows_v1)
        isem = (isem0, isem1)
        osem = (osem0, osem1)

        def idx_start(s, b):
            pltpu.async_copy(
                idx_hbm.at[pl.ds(base + s * _CHUNK, _CHUNK)], idx_b[b], isem[b]
            )

        def idx_wait(b):
            pltpu.make_async_copy(
                idx_hbm.at[pl.ds(base, _CHUNK)], idx_b[b], isem[b]
            ).wait()

        def out_start(s, b):
            b0 = wid * _RG_W + s * _TPC
            pltpu.async_copy(rows_b[b], out_hbm.at[:, pl.ds(b0, _TPC)], osem[b])

        def out_wait(b):
            pltpu.make_async_copy(
                rows_b[b], out_hbm.at[:, pl.ds(wid * _RG_W, _TPC)], osem[b]
            ).wait()

        def compute(idx_ref, rows_ref):
            @plsc.parallel_loop(0, _CHUNK // LANES, unroll=8)
            def group(g):
                t = idx_ref[pl.ds(g * LANES, LANES)]
                j = g // 8
                lane0 = (g % 8) * LANES
                for c in range(DIM):
                    vals = plsc.load_gather(tab_v, [t + c * VOCAB])
                    rows_ref[c // 8, j, c % 8, pl.ds(lane0, LANES)] = vals

        idx_start(0, 0)
        idx_start(1, 1)
        for b in range(2):
            idx_wait(b)
            compute(idx_b[b], rows_b[b])
            out_start(b, b)
            idx_start(b + 2, b)

        def body(si, carry):
            for b in range(2):
                s = 2 * si + b
                idx_wait(b)
                out_wait(b)
                compute(idx_b[b], rows_b[b])
                out_start(s, b)
                idx_start(jnp.minimum(s + 2, _STEPS - 1), b)
            return carry

        lax.fori_loop(1, _STEPS // 2, body, 0)
        for b in range(2):
            out_wait(b)
            idx_wait(b)

    return k


_gather = _make_gather()


def kernel(tokens, embeddings):
    tab_t = embeddings.T.reshape(-1)
    arr4 = _gather(tab_t, tokens)
    return jnp.transpose(arr4, (1, 3, 0, 2)).reshape(N_TOKENS, DIM)


# speedup vs baseline: 4.0645x; 1.0573x over previous
import functools

import jax
import jax.numpy as jnp
from jax import lax
from jax.experimental import pallas as pl
from jax.experimental.pallas import tpu as pltpu
from jax.experimental.pallas import tpu_sc as plsc

N_TOKENS = 3276800
VOCAB = 1000
DIM = 32
LANES = 16

_info = plsc.get_sparse_core_info()
_NC, _NS = _info.num_cores, _info.num_subcores
_NW = _NC * _NS

_B_PER_W = N_TOKENS // _NW
_CHUNK = 1024
_STEPS = _B_PER_W // _CHUNK
_RG = N_TOKENS // 128
_RG_W = _B_PER_W // 128
_CG = DIM // 8
_TPC = _CHUNK // 128


def _make_gather():
    mesh = plsc.VectorSubcoreMesh(core_axis_name="c", subcore_axis_name="s")

    @functools.partial(
        pl.kernel,
        mesh=mesh,
        compiler_params=pltpu.CompilerParams(
            needs_layout_passes=False, use_tc_tiling_on_sc=False
        ),
        out_type=jax.ShapeDtypeStruct((_CG, _RG, 8, 128), jnp.float32),
        scratch_types=[
            pltpu.VMEM((VOCAB * DIM,), jnp.float32),
            pltpu.VMEM((_CHUNK,), jnp.int32),
            pltpu.VMEM((_CHUNK,), jnp.int32),
            pltpu.VMEM((_CG, _TPC, 8, 128), jnp.float32),
            pltpu.VMEM((_CG, _TPC, 8, 128), jnp.float32),
            pltpu.SemaphoreType.DMA,
            pltpu.SemaphoreType.DMA,
            pltpu.SemaphoreType.DMA,
            pltpu.SemaphoreType.DMA,
        ],
    )
    def k(tab_hbm, idx_hbm, out_hbm, tab_v, idx_v0, idx_v1, rows_v0, rows_v1,
          isem0, isem1, osem0, osem1):
        wid = lax.axis_index("s") * _NC + lax.axis_index("c")
        base = wid * _B_PER_W

        pltpu.sync_copy(tab_hbm, tab_v)

        idx_b = (idx_v0, idx_v1)
        rows_b = (rows_v0, rows_v1)
        isem = (isem0, isem1)
        osem = (osem0, osem1)

        def idx_start(s, b):
            pltpu.async_copy(
                idx_hbm.at[pl.ds(base + s * _CHUNK, _CHUNK)], idx_b[b], isem[b]
            )

        def idx_wait(b):
            pltpu.make_async_copy(
                idx_hbm.at[pl.ds(base, _CHUNK)], idx_b[b], isem[b]
            ).wait()

        def out_start(s, b):
            b0 = wid * _RG_W + s * _TPC
            pltpu.async_copy(rows_b[b], out_hbm.at[:, pl.ds(b0, _TPC)], osem[b])

        def out_wait(b):
            pltpu.make_async_copy(
                rows_b[b], out_hbm.at[:, pl.ds(wid * _RG_W, _TPC)], osem[b]
            ).wait()

        def compute(idx_ref, rows_ref):
            @plsc.parallel_loop(0, _CHUNK // LANES, unroll=16)
            def group(g):
                t = idx_ref[pl.ds(g * LANES, LANES)]
                j = g // 8
                lane0 = (g % 8) * LANES
                for c in range(DIM):
                    vals = plsc.load_gather(tab_v, [t + c * VOCAB])
                    rows_ref[c // 8, j, c % 8, pl.ds(lane0, LANES)] = vals

        idx_start(0, 0)
        idx_start(1, 1)
        for b in range(2):
            idx_wait(b)
            compute(idx_b[b], rows_b[b])
            out_start(b, b)
            idx_start(b + 2, b)

        def body(si, carry):
            for b in range(2):
                s = 2 * si + b
                idx_wait(b)
                out_wait(b)
                compute(idx_b[b], rows_b[b])
                out_start(s, b)
                idx_start(jnp.minimum(s + 2, _STEPS - 1), b)
            return carry

        lax.fori_loop(1, _STEPS // 2, body, 0)
        for b in range(2):
            out_wait(b)
            idx_wait(b)

    return k


_gather = _make_gather()


def kernel(tokens, embeddings):
    tab_t = embeddings.T.reshape(-1)
    arr4 = _gather(tab_t, tokens)
    return jnp.transpose(arr4, (1, 3, 0, 2)).reshape(N_TOKENS, DIM)


# speedup vs baseline: 4.2597x; 1.0480x over previous
import functools

import jax
import jax.numpy as jnp
from jax import lax
from jax.experimental import pallas as pl
from jax.experimental.pallas import tpu as pltpu
from jax.experimental.pallas import tpu_sc as plsc

N_TOKENS = 3276800
VOCAB = 1000
DIM = 32
LANES = 16

_info = plsc.get_sparse_core_info()
_NC, _NS = _info.num_cores, _info.num_subcores
_NW = _NC * _NS

_B_PER_W = N_TOKENS // _NW
_CHUNK = 1024
_STEPS = _B_PER_W // _CHUNK
_RG = N_TOKENS // 128
_RG_W = _B_PER_W // 128
_CG = DIM // 8
_TPC = _CHUNK // 128
_CHUNK_W = _CHUNK * DIM
_BLK = _TPC * 8 * 128


def _make_gather():
    mesh = plsc.VectorSubcoreMesh(core_axis_name="c", subcore_axis_name="s")

    @functools.partial(
        pl.kernel,
        mesh=mesh,
        compiler_params=pltpu.CompilerParams(
            needs_layout_passes=False, use_tc_tiling_on_sc=False
        ),
        out_type=jax.ShapeDtypeStruct((_CG * _RG * 8 * 128,), jnp.float32),
        scratch_types=[
            pltpu.VMEM((VOCAB * DIM,), jnp.float32),
            pltpu.VMEM((_CHUNK,), jnp.int32),
            pltpu.VMEM((_CHUNK,), jnp.int32),
            pltpu.VMEM((_CHUNK_W,), jnp.float32),
            pltpu.VMEM((_CHUNK_W,), jnp.float32),
            pltpu.SemaphoreType.DMA,
            pltpu.SemaphoreType.DMA,
            pltpu.SemaphoreType.DMA,
            pltpu.SemaphoreType.DMA,
        ],
    )
    def k(tab_hbm, idx_hbm, out_hbm, tab_v, idx_v0, idx_v1, rows_v0, rows_v1,
          isem0, isem1, osem0, osem1):
        wid = lax.axis_index("s") * _NC + lax.axis_index("c")
        base = wid * _B_PER_W

        pltpu.sync_copy(tab_hbm, tab_v)

        idx_b = (idx_v0, idx_v1)
        rows_b = (rows_v0, rows_v1)
        isem = (isem0, isem1)
        osem = (osem0, osem1)

        iota = lax.iota(jnp.int32, LANES)
        rots = [(iota + k8) & 15 for k8 in range(LANES)]
        svs = [(r >> 3) * (_TPC * 1024) + (r & 7) * 128 + iota for r in rots]

        def idx_start(s, b):
            pltpu.async_copy(
                idx_hbm.at[pl.ds(base + s * _CHUNK, _CHUNK)], idx_b[b], isem[b]
            )

        def idx_wait(b):
            pltpu.make_async_copy(
                idx_hbm.at[pl.ds(base, _CHUNK)], idx_b[b], isem[b]
            ).wait()

        def out_start(s, b):
            b0 = wid * _RG_W + s * _TPC
            for a in range(_CG):
                pltpu.async_copy(
                    rows_b[b].at[pl.ds(a * _BLK, _BLK)],
                    out_hbm.at[pl.ds(a * (_RG * 1024) + b0 * 1024, _BLK)],
                    osem[b],
                )

        def out_wait(b):
            for a in range(_CG):
                pltpu.make_async_copy(
                    rows_b[b].at[pl.ds(a * _BLK, _BLK)],
                    out_hbm.at[pl.ds(a * (_RG * 1024), _BLK)],
                    osem[b],
                ).wait()

        def compute(idx_ref, rows_ref):
            @plsc.parallel_loop(0, _CHUNK // LANES, unroll=16)
            def group(g):
                t32 = idx_ref[pl.ds(g * LANES, LANES)] * DIM
                scal0 = (g // 8) * 1024 + (g % 8) * LANES
                for c0 in (0, 16):
                    gbase = t32 + c0
                    scal = scal0 + (c0 >> 3) * _BLK
                    for k8 in range(LANES):
                        vals = plsc.load_gather(tab_v, [gbase + rots[k8]])
                        plsc.store_scatter(rows_ref, [svs[k8] + scal], vals)

        idx_start(0, 0)
        idx_start(1, 1)
        for b in range(2):
            idx_wait(b)
            compute(idx_b[b], rows_b[b])
            out_start(b, b)
            idx_start(b + 2, b)

        def body(si, carry):
            for b in range(2):
                s = 2 * si + b
                idx_wait(b)
                out_wait(b)
                compute(idx_b[b], rows_b[b])
                out_start(s, b)
                idx_start(jnp.minimum(s + 2, _STEPS - 1), b)
            return carry

        lax.fori_loop(1, _STEPS // 2, body, 0)
        for b in range(2):
            out_wait(b)
            idx_wait(b)

    return k


_gather = _make_gather()


def kernel(tokens, embeddings):
    flat = _gather(embeddings.reshape(-1), tokens)
    arr4 = flat.reshape(_CG, _RG, 8, 128)
    return jnp.transpose(arr4, (1, 3, 0, 2)).reshape(N_TOKENS, DIM)


# speedup vs baseline: 4.8631x; 1.1417x over previous
import functools

import jax
import jax.numpy as jnp
from jax import lax
from jax.experimental import pallas as pl
from jax.experimental.pallas import tpu as pltpu
from jax.experimental.pallas import tpu_sc as plsc

N_TOKENS = 3276800
VOCAB = 1000
DIM = 32
LANES = 16

_info = plsc.get_sparse_core_info()
_NC, _NS = _info.num_cores, _info.num_subcores
_NW = _NC * _NS

_B_PER_W = N_TOKENS // _NW
_CHUNK = 1024
_STEPS = _B_PER_W // _CHUNK
_RG = N_TOKENS // 128
_RG_W = _B_PER_W // 128
_CG = DIM // 8
_TPC = _CHUNK // 128
_CHUNK_W = _CHUNK * DIM
_BLK = _TPC * 8 * 128


def _make_gather():
    mesh = plsc.VectorSubcoreMesh(core_axis_name="c", subcore_axis_name="s")

    @functools.partial(
        pl.kernel,
        mesh=mesh,
        compiler_params=pltpu.CompilerParams(
            needs_layout_passes=False, use_tc_tiling_on_sc=False
        ),
        out_type=jax.ShapeDtypeStruct((_CG * _RG * 8 * 128,), jnp.float32),
        scratch_types=[
            pltpu.VMEM((VOCAB * DIM,), jnp.float32),
            pltpu.VMEM((_CHUNK,), jnp.int32),
            pltpu.VMEM((_CHUNK,), jnp.int32),
            pltpu.VMEM((_CHUNK_W,), jnp.float32),
            pltpu.VMEM((_CHUNK_W,), jnp.float32),
            pltpu.SemaphoreType.DMA,
            pltpu.SemaphoreType.DMA,
            pltpu.SemaphoreType.DMA,
            pltpu.SemaphoreType.DMA,
        ],
    )
    def k(tab_hbm, idx_hbm, out_hbm, tab_v, idx_v0, idx_v1, rows_v0, rows_v1,
          isem0, isem1, osem0, osem1):
        wid = lax.axis_index("s") * _NC + lax.axis_index("c")
        base = wid * _B_PER_W

        pltpu.sync_copy(tab_hbm, tab_v)

        idx_b = (idx_v0, idx_v1)
        rows_b = (rows_v0, rows_v1)
        isem = (isem0, isem1)
        osem = (osem0, osem1)

        iota = lax.iota(jnp.int32, LANES)
        rots = [(iota + k8) & 15 for k8 in range(LANES)]
        svs = [(r >> 3) * (_TPC * 1024) + (r & 7) * 128 + iota for r in rots]

        def idx_start(s, b):
            pltpu.async_copy(
                idx_hbm.at[pl.ds(base + s * _CHUNK, _CHUNK)], idx_b[b], isem[b]
            )

        def idx_wait(b):
            pltpu.make_async_copy(
                idx_hbm.at[pl.ds(base, _CHUNK)], idx_b[b], isem[b]
            ).wait()

        def out_start(s, b):
            b0 = wid * _RG_W + s * _TPC
            for a in range(_CG):
                pltpu.async_copy(
                    rows_b[b].at[pl.ds(a * _BLK, _BLK)],
                    out_hbm.at[pl.ds(a * (_RG * 1024) + b0 * 1024, _BLK)],
                    osem[b],
                )

        def out_wait(b):
            for a in range(_CG):
                pltpu.make_async_copy(
                    rows_b[b].at[pl.ds(a * _BLK, _BLK)],
                    out_hbm.at[pl.ds(a * (_RG * 1024), _BLK)],
                    osem[b],
                ).wait()

        def compute(idx_ref, rows_ref):
            @plsc.parallel_loop(0, _CHUNK // LANES, unroll=16)
            def group(g):
                t32 = idx_ref[pl.ds(g * LANES, LANES)] * DIM
                scal0 = (g // 8) * 1024 + (g % 8) * LANES
                for c0 in (0, 16):
                    gbase = t32 + c0
                    scal = scal0 + (c0 >> 3) * _BLK
                    win = rows_ref.at[pl.ds(scal, _BLK + 7 * 128 + LANES)]
                    for k8 in range(LANES):
                        vals = plsc.load_gather(tab_v, [gbase + rots[k8]])
                        plsc.store_scatter(win, [svs[k8]], vals)

        idx_start(0, 0)
        idx_start(1, 1)
        for b in range(2):
            idx_wait(b)
            compute(idx_b[b], rows_b[b])
            out_start(b, b)
            idx_start(b + 2, b)

        def body(si, carry):
            for b in range(2):
                s = 2 * si + b
                idx_wait(b)
                out_wait(b)
                compute(idx_b[b], rows_b[b])
                out_start(s, b)
                idx_start(jnp.minimum(s + 2, _STEPS - 1), b)
            return carry

        lax.fori_loop(1, _STEPS // 2, body, 0)
        for b in range(2):
            out_wait(b)
            idx_wait(b)

    return k


_gather = _make_gather()


def kernel(tokens, embeddings):
    flat = _gather(embeddings.reshape(-1), tokens)
    arr4 = flat.reshape(_CG, _RG, 8, 128)
    return jnp.transpose(arr4, (1, 3, 0, 2)).reshape(N_TOKENS, DIM)
